# baseline (speedup 1.0000x reference)
import functools

import jax
import jax.numpy as jnp
from jax import lax
from jax.experimental import pallas as pl
from jax.experimental.pallas import tpu as pltpu, tpu_sc as plsc

_KNN = 16
_BIG = 3.0e38
_TW = 256
_CH = 256


def _setup_kernel(x_ref, pos_ref, wqkv_ref, w1_ref, q_ref, pw_ref, key_ref,
                  tbl_ref):
    f32 = jnp.float32
    dot_hi = functools.partial(jax.lax.dot_general, preferred_element_type=f32,
                               precision=jax.lax.Precision.HIGHEST)
    pos = pos_ref[0]
    x = x_ref[0]
    n, d = x.shape

    qkv = dot_hi(x, wqkv_ref[...], (((1,), (0,)), ((), ())))
    q_ref[0] = qkv[:, :d]
    pw = dot_hi(pos, w1_ref[...], (((1,), (0,)), ((), ())))
    pw_ref[0] = pw

    pad = jnp.zeros((n, _TW - 3 * d), f32)
    tbl_ref[0] = jnp.concatenate([qkv[:, d:], pw, pad], axis=1)

    ones = jnp.ones((n, 1), f32)
    outer = (((1,), (1,)), ((), ()))
    d2 = None
    for c in range(3):
        xc = pos[:, c:c + 1]
        rel_c = dot_hi(xc, ones, outer) - dot_hi(ones, xc, outer)
        sq = rel_c * rel_c
        d2 = sq if d2 is None else d2 + sq
    key_ref[0] = jnp.sqrt(d2)


def _select_kernel(key_in_ref, idx_ref, th_ref, li_ref):
    i = pl.program_id(0)
    t = pl.program_id(1)
    n = key_in_ref.shape[1]
    first = t == 0
    key = key_in_ref[0]

    th = jnp.where(first, -_BIG, th_ref[...])
    li = jnp.where(first, -1, li_ref[...])
    iota_j = lax.broadcasted_iota(jnp.int32, key.shape, 1)
    eligible = (key > th) | ((key == th) & (iota_j > li))
    keyx = jnp.where(eligible, key, _BIG)
    rmin = jnp.min(keyx, axis=1, keepdims=True)
    cand = jnp.where(keyx == rmin, iota_j, n)
    amin = jnp.min(cand, axis=1, keepdims=True)
    th_ref[...] = rmin
    li_ref[...] = amin
    idx_ref[0, 0] = amin + i * n


def _mlp_kernel(g_ref, q_ref, pw_ref, b1_ref, w2_ref, b2_ref, a1_ref,
                ab1_ref, a2_ref, ab2_ref, out_ref, m_ref, s_ref, acc_ref):
    t = pl.program_id(1)
    d = q_ref.shape[2]
    first = t == 0

    f32 = jnp.float32
    dot = functools.partial(jax.lax.dot_general, preferred_element_type=f32,
                            precision=jax.lax.Precision.HIGHEST)
    cdims = (((1,), (0,)), ((), ()))

    g = g_ref[0, 0]
    k_g = g[:, :d]
    v_g = g[:, d:2 * d]
    pw_g = g[:, 2 * d:3 * d]

    h = jax.nn.relu(pw_ref[0] - pw_g + b1_ref[...])
    pe = dot(h, w2_ref[...], cdims) + b2_ref[...]

    s_in = q_ref[0] - k_g + pe
    h2 = jax.nn.relu(dot(s_in, a1_ref[...], cdims) + ab1_ref[...])
    sim = dot(h2, a2_ref[...], cdims) + ab2_ref[...]
    v2 = v_g + pe

    m = jnp.where(first, -_BIG, m_ref[...])
    s = jnp.where(first, 0.0, s_ref[...])
    acc = jnp.where(first, 0.0, acc_ref[...])
    m_new = jnp.maximum(m, sim)
    c = jnp.exp(m - m_new)
    p = jnp.exp(sim - m_new)
    s_new = s * c + p
    acc_new = acc * c + p * v2
    m_ref[...] = m_new
    s_ref[...] = s_new
    acc_ref[...] = acc_new

    out_ref[0] = acc_new / s_new


def _make_sc_gather(width, total):
    info = plsc.get_sparse_core_info()
    nc = info.num_cores
    nw = nc * info.num_subcores
    b_per_w = total // nw
    mesh = plsc.VectorSubcoreMesh(core_axis_name="c", subcore_axis_name="s")

    @functools.partial(
        pl.kernel, mesh=mesh,
        out_type=jax.ShapeDtypeStruct((total, width), jnp.float32),
        scratch_types=[
            pltpu.VMEM((_CH,), jnp.int32),
            pltpu.VMEM((_CH, width), jnp.float32),
            pltpu.SemaphoreType.DMA,
        ],
    )
    def sc_gather(table_hbm, idx_hbm, out_hbm, idx_v, rows_v, sem):
        wid = lax.axis_index("s") * nc + lax.axis_index("c")
        for g in range(b_per_w // _CH):
            base = wid * b_per_w + g * _CH
            pltpu.sync_copy(idx_hbm.at[pl.ds(base, _CH)], idx_v)
            pltpu.async_copy(table_hbm.at[idx_v], rows_v, sem).wait()
            pltpu.sync_copy(rows_v, out_hbm.at[pl.ds(base, _CH)])

    return sc_gather


def kernel(x, pos, Wqkv, W1, b1, W2, b2, A1, ab1, A2, ab2):
    b, n, d = x.shape

    q, pw, key, tbl = pl.pallas_call(
        _setup_kernel,
        grid=(b,),
        in_specs=[
            pl.BlockSpec((1, n, d), lambda i: (i, 0, 0)),
            pl.BlockSpec((1, n, 3), lambda i: (i, 0, 0)),
            pl.BlockSpec(Wqkv.shape, lambda i: (0, 0)),
            pl.BlockSpec(W1.shape, lambda i: (0, 0)),
        ],
        out_specs=[
            pl.BlockSpec((1, n, d), lambda i: (i, 0, 0)),
            pl.BlockSpec((1, n, d), lambda i: (i, 0, 0)),
            pl.BlockSpec((1, n, n), lambda i: (i, 0, 0)),
            pl.BlockSpec((1, n, _TW), lambda i: (i, 0, 0)),
        ],
        out_shape=[
            jax.ShapeDtypeStruct((b, n, d), jnp.float32),
            jax.ShapeDtypeStruct((b, n, d), jnp.float32),
            jax.ShapeDtypeStruct((b, n, n), jnp.float32),
            jax.ShapeDtypeStruct((b, n, _TW), jnp.float32),
        ],
    )(x, pos, Wqkv, W1)

    idx = pl.pallas_call(
        _select_kernel,
        grid=(b, _KNN),
        in_specs=[pl.BlockSpec((1, n, n), lambda i, t: (i, 0, 0))],
        out_specs=pl.BlockSpec((1, 1, n, 1), lambda i, t: (i, t, 0, 0)),
        out_shape=jax.ShapeDtypeStruct((b, _KNN, n, 1), jnp.int32),
        scratch_shapes=[
            pltpu.VMEM((n, 1), jnp.float32),
            pltpu.VMEM((n, 1), jnp.int32),
        ],
        compiler_params=pltpu.CompilerParams(
            dimension_semantics=("arbitrary", "arbitrary"),
        ),
    )(key)

    total = b * _KNN * n
    gathered = _make_sc_gather(_TW, total)(
        tbl.reshape(b * n, _TW), idx.reshape(total))
    g4 = gathered.reshape(b, _KNN, n, _TW)

    full = lambda a: pl.BlockSpec(a.shape, lambda i, t: (0,) * a.ndim)
    row = lambda a: pl.BlockSpec((1, a.shape[0]), lambda i, t: (0, 0))
    bat = lambda w: pl.BlockSpec((1, n, w), lambda i, t: (i, 0, 0))
    return pl.pallas_call(
        _mlp_kernel,
        grid=(b, _KNN),
        in_specs=[
            pl.BlockSpec((1, 1, n, _TW), lambda i, t: (i, t, 0, 0)),
            bat(d), bat(d),
            row(b1), full(W2), row(b2),
            full(A1), row(ab1), full(A2), row(ab2),
        ],
        out_specs=pl.BlockSpec((1, n, d), lambda i, t: (i, 0, 0)),
        out_shape=jax.ShapeDtypeStruct((b, n, d), jnp.float32),
        scratch_shapes=[
            pltpu.VMEM((n, d), jnp.float32),
            pltpu.VMEM((n, d), jnp.float32),
            pltpu.VMEM((n, d), jnp.float32),
        ],
        compiler_params=pltpu.CompilerParams(
            dimension_semantics=("arbitrary", "arbitrary"),
        ),
    )(g4, q, pw, b1.reshape(1, -1), W2, b2.reshape(1, -1),
      A1, ab1.reshape(1, -1), A2, ab2.reshape(1, -1))
